# baseline (speedup 1.0000x reference)
import jax
import jax.numpy as jnp
from jax.experimental import pallas as pl
from jax.experimental.pallas import tpu as pltpu

K = 8
TB = 8


def _body(idx_ref, x_ref, w_ref, *rest):
    rows = rest[: TB * K]
    out_ref = rest[TB * K]
    for i in range(TB):
        xi = x_ref[0, i : i + 1, :]
        acc = xi
        for k in range(K):
            r = rows[i * K + k][0]
            p = jnp.sum(xi * r)
            acc = acc + (jnp.tanh(p) * w_ref[i, k]) * r
        out_ref[0, i : i + 1, :] = acc


def kernel(x, indices, weights, table):
    tokens, d = x.shape
    idx_flat = indices.astype(jnp.int32).reshape(-1)
    nblk = tokens // TB
    x3 = x.reshape(nblk, TB, d)
    table3 = table.reshape(table.shape[0], 1, d)

    def row_spec(j):
        return pl.BlockSpec((1, 1, d), lambda t, idx, _j=j: (idx[t * (TB * K) + _j], 0, 0))

    grid_spec = pltpu.PrefetchScalarGridSpec(
        num_scalar_prefetch=1,
        grid=(nblk,),
        in_specs=[pl.BlockSpec((1, TB, d), lambda t, idx: (t, 0, 0)),
                  pl.BlockSpec((TB, K), lambda t, idx: (t, 0))]
        + [row_spec(j) for j in range(TB * K)],
        out_specs=pl.BlockSpec((1, TB, d), lambda t, idx: (t, 0, 0)),
    )
    f = pl.pallas_call(
        _body,
        grid_spec=grid_spec,
        out_shape=jax.ShapeDtypeStruct((nblk, TB, d), x.dtype),
    )
    return f(idx_flat, x3, weights, *([table3] * (TB * K))).reshape(tokens, d)


# speedup vs baseline: 10.8620x; 10.8620x over previous
import dataclasses
import functools

import jax
import jax.numpy as jnp
from jax import lax
from jax.experimental import pallas as pl
from jax.experimental.pallas import tpu as pltpu
from jax.experimental.pallas import tpu_sc as plsc

TOKENS = 16384
D = 1024
K = 8
L = 16
NW = 32
TPW = TOKENS // NW
G = 8
NG = TPW // G
NC = D // L


def _sc_kernel(x_hbm, idx_hbm, w_hbm, tbl_hbm, out_hbm,
               idx_v, w_v, rows_v, x_v, o_v, sem):
    wid = lax.axis_index("s") * 2 + lax.axis_index("c")
    t0 = wid * TPW

    pltpu.sync_copy(idx_hbm.at[pl.ds(t0 * K, TPW * K)], idx_v)
    pltpu.sync_copy(w_hbm.at[pl.ds(t0 * K, TPW * K)], w_v)

    @pl.loop(0, NG)
    def _group(g):
        tok = t0 + g * G
        pltpu.sync_copy(x_hbm.at[pl.ds(tok, G)], x_v)
        pltpu.async_copy(
            tbl_hbm.at[idx_v.at[pl.ds(g * (G * K), G * K)]], rows_v, sem
        ).wait()

        wvecs = [w_v[pl.ds(g * (G * K) + j * L, L)] for j in range(G * K // L)]

        for i in range(G):
            def dot_body(c, accs, _i=i):
                xc = x_v[_i, pl.ds(c * L, L)]
                return tuple(
                    accs[k] + xc * rows_v[_i * K + k, pl.ds(c * L, L)]
                    for k in range(K)
                )

            accs = lax.fori_loop(
                0, NC, dot_body,
                tuple(jnp.zeros((L,), jnp.float32) for _ in range(K)),
            )

            weff = []
            for k in range(K):
                p = jnp.sum(accs[k])
                pv = jnp.full((L,), p, jnp.float32)
                e = jnp.exp(-2.0 * jnp.abs(pv))
                th = jnp.sign(pv) * (1.0 - e) / (1.0 + e)
                j = i * K + k
                weff.append(th * wvecs[j // L][j % L])

            def comb_body(c, carry, _i=i, _weff=weff):
                s = pl.ds(c * L, L)
                acc = x_v[_i, s]
                for k in range(K):
                    acc = acc + _weff[k] * rows_v[_i * K + k, s]
                o_v[_i, s] = acc
                return carry

            lax.fori_loop(0, NC, comb_body, 0)

        pltpu.sync_copy(o_v, out_hbm.at[pl.ds(tok, G)])


def kernel(x, indices, weights, table):
    idx_flat = indices.astype(jnp.int32).reshape(-1)
    w_flat = weights.reshape(-1)
    mesh = plsc.VectorSubcoreMesh(core_axis_name="c", subcore_axis_name="s")
    cp = pltpu.CompilerParams()
    if "needs_layout_passes" in pltpu.CompilerParams.__dataclass_fields__:
        cp = dataclasses.replace(cp, needs_layout_passes=False)
    f = pl.kernel(
        _sc_kernel,
        mesh=mesh,
        compiler_params=cp,
        out_type=jax.ShapeDtypeStruct((TOKENS, D), jnp.float32),
        scratch_types=[
            pltpu.VMEM((TPW * K,), jnp.int32),
            pltpu.VMEM((TPW * K,), jnp.float32),
            pltpu.VMEM((G * K, D), jnp.float32),
            pltpu.VMEM((G, D), jnp.float32),
            pltpu.VMEM((G, D), jnp.float32),
            pltpu.SemaphoreType.DMA,
        ],
    )
    return f(x, idx_flat, w_flat, table)
